# baseline (speedup 1.0000x reference)
import functools
import math

import jax
import jax.numpy as jnp
import numpy as np
from jax import lax
from jax.experimental import pallas as pl
from jax.experimental.pallas import tpu as pltpu
from jax.experimental.pallas import tpu_sc as plsc

HID = 1024
HEAD_DIM = 64
BASE = 10000.0
EPS = 1e-12

NC = 2
NS = 16
NW = NC * NS
L = 16

TOK = 8192
TPW = TOK // NW
CH = 32
NCH = TPW // CH
NJ = HID // L

_RSQRT_MAGIC = 0x5F3759DF


def _lane_sum(x):
    dnums = lax.GatherDimensionNumbers(
        offset_dims=(), collapsed_slice_dims=(0,), start_index_map=(0,)
    )
    lane = lax.iota(jnp.int32, L)
    for sh in (8, 4, 2, 1):
        perm = jnp.reshape((lane + sh) & (L - 1), (L, 1))
        x = x + lax.gather(
            x, perm, dnums, (1,), mode=lax.GatherScatterMode.PROMISE_IN_BOUNDS
        )
    return x


def _rsqrt_newton(va):
    bits = lax.bitcast_convert_type(va, jnp.int32)
    y = lax.bitcast_convert_type(_RSQRT_MAGIC - (bits >> 1), jnp.float32)
    for _ in range(3):
        y = y * (1.5 - 0.5 * va * y * y)
    return y


_sc_mesh = plsc.VectorSubcoreMesh(
    core_axis_name="c", subcore_axis_name="s", num_cores=NC, num_subcores=NS
)


@functools.partial(
    pl.kernel,
    out_type=jax.ShapeDtypeStruct((TOK, HID), jnp.float32),
    mesh=_sc_mesh,
    scratch_types=[
        pltpu.VMEM((NCH, CH), jnp.int32),
        pltpu.VMEM((CH, HID), jnp.float32),
        pltpu.VMEM((HID,), jnp.float32),
        pltpu.VMEM((HID,), jnp.float32),
        pltpu.VMEM((HID,), jnp.float32),
        pltpu.SemaphoreType.DMA,
    ],
)
def _emb_ln_sc(ids_hbm, table_hbm, tt_hbm, g_hbm, b_hbm, out_hbm,
               idx_v, buf, tt_v, g_v, b_v, sem):
    wid = lax.axis_index("s") * NC + lax.axis_index("c")
    pltpu.sync_copy(ids_hbm.at[wid], idx_v)
    pltpu.sync_copy(tt_hbm, tt_v)
    pltpu.sync_copy(g_hbm, g_v)
    pltpu.sync_copy(b_hbm, b_v)

    def chunk_body(c, carry):
        pltpu.async_copy(table_hbm.at[idx_v.at[c]], buf, sem).wait()

        def row_body(r, rcarry):
            acc = jnp.zeros((L,), jnp.float32)
            acc2 = jnp.zeros((L,), jnp.float32)
            for j in range(NJ):
                sl = pl.ds(j * L, L)
                x = buf[r, sl] + tt_v[sl]
                buf[r, sl] = x
                acc = acc + x
                acc2 = acc2 + x * x
            muv = _lane_sum(acc) * (1.0 / HID)
            varv = _lane_sum(acc2) * (1.0 / HID) - muv * muv
            inv = _rsqrt_newton(varv + EPS)
            for j in range(NJ):
                sl = pl.ds(j * L, L)
                x = buf[r, sl]
                buf[r, sl] = (x - muv) * inv * g_v[sl] + b_v[sl]
            return rcarry

        lax.fori_loop(0, CH, row_body, 0)
        pltpu.sync_copy(buf, out_hbm.at[pl.ds(wid * TPW + c * CH, CH)])
        return carry

    lax.fori_loop(0, NCH, chunk_body, 0)


def _rope_body(cos_ref, sin_ref):
    s_len, d = cos_ref.shape
    pos = lax.broadcasted_iota(jnp.int32, (s_len, d), 0).astype(jnp.float32)
    ch = lax.broadcasted_iota(jnp.int32, (s_len, d), 1)
    half = d // 2
    i = jnp.where(ch < half, ch, ch - half).astype(jnp.float32)
    inv_freq = jnp.exp(i * (-2.0 * math.log(BASE) / d))
    ang = pos * inv_freq
    cos_ref[...] = jnp.cos(ang)
    sin_ref[...] = jnp.sin(ang)


def kernel(input_ids, word_emb, token_type_emb, ln_gamma, ln_beta):
    b, s = input_ids.shape
    ids = input_ids.reshape(NW, NCH, CH).astype(jnp.int32)
    tt0 = token_type_emb[0]

    emb_flat = _emb_ln_sc(ids, word_emb, tt0, ln_gamma, ln_beta)
    embeddings = emb_flat.reshape(b, s, HID)

    cos_c, sin_c = pl.pallas_call(
        _rope_body,
        out_shape=(
            jax.ShapeDtypeStruct((s, HEAD_DIM), jnp.float32),
            jax.ShapeDtypeStruct((s, HEAD_DIM), jnp.float32),
        ),
    )()
    rope_cos = jnp.broadcast_to(cos_c[None, :, None, :], (b, s, 1, HEAD_DIM))
    rope_sin = jnp.broadcast_to(sin_c[None, :, None, :], (b, s, 1, HEAD_DIM))

    attention_mask = jnp.ones((b, s), dtype=jnp.float32)
    return embeddings, attention_mask, rope_cos, rope_sin


# speedup vs baseline: 2.7179x; 2.7179x over previous
import functools
import math

import jax
import jax.numpy as jnp
from jax import lax
from jax.experimental import pallas as pl
from jax.experimental.pallas import tpu as pltpu
from jax.experimental.pallas import tpu_sc as plsc

HID = 1024
HEAD_DIM = 64
BASE = 10000.0
EPS = 1e-12

NC = 2
NS = 16
NW = NC * NS

TOK = 8192
TPW = TOK // NW
CH = 32
NCH = TPW // CH

_sc_mesh = plsc.VectorSubcoreMesh(
    core_axis_name="c", subcore_axis_name="s", num_cores=NC, num_subcores=NS
)


@functools.partial(
    pl.kernel,
    out_type=jax.ShapeDtypeStruct((TOK, HID), jnp.float32),
    mesh=_sc_mesh,
    scratch_types=[
        pltpu.VMEM((NCH, CH), jnp.int32),
        pltpu.VMEM((CH, HID), jnp.float32),
        pltpu.VMEM((CH, HID), jnp.float32),
        pltpu.SemaphoreType.DMA,
        pltpu.SemaphoreType.DMA,
    ],
)
def _gather_sc(ids_hbm, table_hbm, out_hbm, idx_v, buf_a, buf_b, sem_a, sem_b):
    wid = lax.axis_index("s") * NC + lax.axis_index("c")
    pltpu.sync_copy(ids_hbm.at[wid], idx_v)
    bufs = (buf_a, buf_b)
    sems = (sem_a, sem_b)

    def start(c):
        pltpu.make_async_copy(
            table_hbm.at[idx_v.at[c]], bufs[c % 2], sems[c % 2]
        ).start()

    def wait(c):
        pltpu.make_async_copy(
            table_hbm.at[idx_v.at[c]], bufs[c % 2], sems[c % 2]
        ).wait()

    start(0)
    for c in range(NCH):
        if c + 1 < NCH:
            start(c + 1)
        wait(c)
        pltpu.sync_copy(bufs[c % 2], out_hbm.at[pl.ds(wid * TPW + c * CH, CH)])


def _ln_body(rows_ref, tt_ref, g_ref, b_ref, out_ref):
    x = rows_ref[...] + tt_ref[...]
    mu = jnp.mean(x, axis=1, keepdims=True)
    xc = x - mu
    var = jnp.mean(xc * xc, axis=1, keepdims=True)
    out_ref[...] = xc * lax.rsqrt(var + EPS) * g_ref[...] + b_ref[...]


TB = 256


def _ln_tc(rows, tt0, gamma, beta):
    return pl.pallas_call(
        _ln_body,
        grid=(TOK // TB,),
        in_specs=[
            pl.BlockSpec((TB, HID), lambda i: (i, 0)),
            pl.BlockSpec((1, HID), lambda i: (0, 0)),
            pl.BlockSpec((1, HID), lambda i: (0, 0)),
            pl.BlockSpec((1, HID), lambda i: (0, 0)),
        ],
        out_specs=pl.BlockSpec((TB, HID), lambda i: (i, 0)),
        out_shape=jax.ShapeDtypeStruct((TOK, HID), jnp.float32),
    )(rows, tt0.reshape(1, HID), gamma.reshape(1, HID), beta.reshape(1, HID))


def _rope_body(cos_ref, sin_ref):
    s_len, d = cos_ref.shape
    pos = lax.broadcasted_iota(jnp.int32, (s_len, d), 0).astype(jnp.float32)
    ch = lax.broadcasted_iota(jnp.int32, (s_len, d), 1)
    half = d // 2
    i = jnp.where(ch < half, ch, ch - half).astype(jnp.float32)
    inv_freq = jnp.exp(i * (-2.0 * math.log(BASE) / d))
    ang = pos * inv_freq
    cos_ref[...] = jnp.cos(ang)
    sin_ref[...] = jnp.sin(ang)


def kernel(input_ids, word_emb, token_type_emb, ln_gamma, ln_beta):
    b, s = input_ids.shape
    ids = input_ids.reshape(NW, NCH, CH).astype(jnp.int32)
    tt0 = token_type_emb[0]

    rows = _gather_sc(ids, word_emb)
    emb_flat = _ln_tc(rows, tt0, ln_gamma, ln_beta)
    embeddings = emb_flat.reshape(b, s, HID)

    cos_c, sin_c = pl.pallas_call(
        _rope_body,
        out_shape=(
            jax.ShapeDtypeStruct((s, HEAD_DIM), jnp.float32),
            jax.ShapeDtypeStruct((s, HEAD_DIM), jnp.float32),
        ),
    )()
    rope_cos = jnp.broadcast_to(cos_c[None, :, None, :], (b, s, 1, HEAD_DIM))
    rope_sin = jnp.broadcast_to(sin_c[None, :, None, :], (b, s, 1, HEAD_DIM))

    attention_mask = jnp.ones((b, s), dtype=jnp.float32)
    return embeddings, attention_mask, rope_cos, rope_sin


# speedup vs baseline: 3.0435x; 1.1198x over previous
import functools
import math

import jax
import jax.numpy as jnp
from jax import lax
from jax.experimental import pallas as pl
from jax.experimental.pallas import tpu as pltpu
from jax.experimental.pallas import tpu_sc as plsc

HID = 1024
HEAD_DIM = 64
BASE = 10000.0
EPS = 1e-12

NC = 2
NS = 16
NW = NC * NS

TOK = 8192
TPW = TOK // NW
CH = 32
NCH = TPW // CH

_sc_mesh = plsc.VectorSubcoreMesh(
    core_axis_name="c", subcore_axis_name="s", num_cores=NC, num_subcores=NS
)


@functools.partial(
    pl.kernel,
    out_type=jax.ShapeDtypeStruct((TOK, HID), jnp.float32),
    mesh=_sc_mesh,
    scratch_types=[
        pltpu.VMEM((NCH, CH), jnp.int32),
        pltpu.VMEM((CH, HID), jnp.float32),
        pltpu.VMEM((CH, HID), jnp.float32),
        pltpu.SemaphoreType.DMA,
        pltpu.SemaphoreType.DMA,
    ],
)
def _gather_sc(ids_hbm, table_hbm, out_hbm, idx_v, buf_a, buf_b, sem_a, sem_b):
    wid = lax.axis_index("s") * NC + lax.axis_index("c")
    pltpu.sync_copy(ids_hbm.at[wid], idx_v)
    bufs = (buf_a, buf_b)
    sems = (sem_a, sem_b)

    def start(c):
        pltpu.make_async_copy(
            table_hbm.at[idx_v.at[c]], bufs[c % 2], sems[c % 2]
        ).start()

    def wait(c):
        pltpu.make_async_copy(
            table_hbm.at[idx_v.at[c]], bufs[c % 2], sems[c % 2]
        ).wait()

    start(0)
    for c in range(NCH):
        if c + 1 < NCH:
            start(c + 1)
        wait(c)
        pltpu.sync_copy(bufs[c % 2], out_hbm.at[pl.ds(wid * TPW + c * CH, CH)])


def _ln_body(rows_ref, tt_ref, g_ref, b_ref, out_ref):
    x = rows_ref[...] + tt_ref[...]
    mu = jnp.mean(x, axis=1, keepdims=True)
    xc = x - mu
    var = jnp.mean(xc * xc, axis=1, keepdims=True)
    out_ref[...] = xc * lax.rsqrt(var + EPS) * g_ref[...] + b_ref[...]


TB = 512


def _ln_tc(rows, tt0, gamma, beta):
    return pl.pallas_call(
        _ln_body,
        grid=(TOK // TB,),
        in_specs=[
            pl.BlockSpec((TB, HID), lambda i: (i, 0)),
            pl.BlockSpec((1, HID), lambda i: (0, 0)),
            pl.BlockSpec((1, HID), lambda i: (0, 0)),
            pl.BlockSpec((1, HID), lambda i: (0, 0)),
        ],
        out_specs=pl.BlockSpec((TB, HID), lambda i: (i, 0)),
        out_shape=jax.ShapeDtypeStruct((TOK, HID), jnp.float32),
    )(rows, tt0.reshape(1, HID), gamma.reshape(1, HID), beta.reshape(1, HID))


def _rope_body(cos_ref, sin_ref):
    s_len, d = cos_ref.shape
    half = d // 2
    pos = lax.broadcasted_iota(jnp.int32, (s_len, half), 0).astype(jnp.float32)
    i = lax.broadcasted_iota(jnp.int32, (s_len, half), 1).astype(jnp.float32)
    inv_freq = jnp.exp(i * (-2.0 * math.log(BASE) / d))
    ang = pos * inv_freq
    c = jnp.cos(ang)
    s = jnp.sin(ang)
    cos_ref[:, :half] = c
    cos_ref[:, half:] = c
    sin_ref[:, :half] = s
    sin_ref[:, half:] = s


def kernel(input_ids, word_emb, token_type_emb, ln_gamma, ln_beta):
    b, s = input_ids.shape
    ids = input_ids.reshape(NW, NCH, CH).astype(jnp.int32)
    tt0 = token_type_emb[0]

    rows = _gather_sc(ids, word_emb)
    emb_flat = _ln_tc(rows, tt0, ln_gamma, ln_beta)
    embeddings = emb_flat.reshape(b, s, HID)

    cos_c, sin_c = pl.pallas_call(
        _rope_body,
        out_shape=(
            jax.ShapeDtypeStruct((s, HEAD_DIM), jnp.float32),
            jax.ShapeDtypeStruct((s, HEAD_DIM), jnp.float32),
        ),
    )()
    rope_cos = jnp.broadcast_to(cos_c[None, :, None, :], (b, s, 1, HEAD_DIM))
    rope_sin = jnp.broadcast_to(sin_c[None, :, None, :], (b, s, 1, HEAD_DIM))

    attention_mask = jnp.ones((b, s), dtype=jnp.float32)
    return embeddings, attention_mask, rope_cos, rope_sin


# speedup vs baseline: 3.1755x; 1.0434x over previous
import functools
import math

import jax
import jax.numpy as jnp
from jax import lax
from jax.experimental import pallas as pl
from jax.experimental.pallas import tpu as pltpu
from jax.experimental.pallas import tpu_sc as plsc

HID = 1024
HEAD_DIM = 64
BASE = 10000.0
EPS = 1e-12

NC = 2
NS = 16
NW = NC * NS

TOK = 8192
TPW = TOK // NW
CH = 32
NCH = TPW // CH

_sc_mesh = plsc.VectorSubcoreMesh(
    core_axis_name="c", subcore_axis_name="s", num_cores=NC, num_subcores=NS
)


@functools.partial(
    pl.kernel,
    out_type=jax.ShapeDtypeStruct((TOK, HID), jnp.float32),
    mesh=_sc_mesh,
    scratch_types=[
        pltpu.VMEM((NCH, CH), jnp.int32),
        pltpu.VMEM((CH, HID), jnp.float32),
        pltpu.VMEM((CH, HID), jnp.float32),
        pltpu.SemaphoreType.DMA,
        pltpu.SemaphoreType.DMA,
    ],
)
def _gather_sc(ids_hbm, table_hbm, out_hbm, idx_v, buf_a, buf_b, sem_a, sem_b):
    wid = lax.axis_index("s") * NC + lax.axis_index("c")
    pltpu.sync_copy(ids_hbm.at[wid], idx_v)
    bufs = (buf_a, buf_b)
    sems = (sem_a, sem_b)

    def start(c):
        pltpu.make_async_copy(
            table_hbm.at[idx_v.at[c]], bufs[c % 2], sems[c % 2]
        ).start()

    def wait(c):
        pltpu.make_async_copy(
            table_hbm.at[idx_v.at[c]], bufs[c % 2], sems[c % 2]
        ).wait()

    start(0)
    for c in range(NCH):
        if c + 1 < NCH:
            start(c + 1)
        wait(c)
        pltpu.sync_copy(bufs[c % 2], out_hbm.at[pl.ds(wid * TPW + c * CH, CH)])


def _ln_body(rows_ref, tt_ref, g_ref, b_ref, out_ref):
    x = rows_ref[...] + tt_ref[...]
    mu = jnp.mean(x, axis=1, keepdims=True)
    xc = x - mu
    var = jnp.mean(xc * xc, axis=1, keepdims=True)
    out_ref[...] = xc * lax.rsqrt(var + EPS) * g_ref[...] + b_ref[...]


TB = 1024


def _ln_tc(rows, tt0, gamma, beta):
    return pl.pallas_call(
        _ln_body,
        grid=(TOK // TB,),
        in_specs=[
            pl.BlockSpec((TB, HID), lambda i: (i, 0)),
            pl.BlockSpec((1, HID), lambda i: (0, 0)),
            pl.BlockSpec((1, HID), lambda i: (0, 0)),
            pl.BlockSpec((1, HID), lambda i: (0, 0)),
        ],
        out_specs=pl.BlockSpec((TB, HID), lambda i: (i, 0)),
        out_shape=jax.ShapeDtypeStruct((TOK, HID), jnp.float32),
    )(rows, tt0.reshape(1, HID), gamma.reshape(1, HID), beta.reshape(1, HID))


def _rope_body(cos_ref, sin_ref):
    s_len, d = cos_ref.shape
    half = d // 2
    pos = lax.broadcasted_iota(jnp.int32, (s_len, half), 0).astype(jnp.float32)
    i = lax.broadcasted_iota(jnp.int32, (s_len, half), 1).astype(jnp.float32)
    inv_freq = jnp.exp(i * (-2.0 * math.log(BASE) / d))
    ang = pos * inv_freq
    c = jnp.cos(ang)
    s = jnp.sin(ang)
    cos_ref[:, :half] = c
    cos_ref[:, half:] = c
    sin_ref[:, :half] = s
    sin_ref[:, half:] = s


def kernel(input_ids, word_emb, token_type_emb, ln_gamma, ln_beta):
    b, s = input_ids.shape
    ids = input_ids.reshape(NW, NCH, CH).astype(jnp.int32)
    tt0 = token_type_emb[0]

    rows = _gather_sc(ids, word_emb)
    emb_flat = _ln_tc(rows, tt0, ln_gamma, ln_beta)
    embeddings = emb_flat.reshape(b, s, HID)

    cos_c, sin_c = pl.pallas_call(
        _rope_body,
        out_shape=(
            jax.ShapeDtypeStruct((s, HEAD_DIM), jnp.float32),
            jax.ShapeDtypeStruct((s, HEAD_DIM), jnp.float32),
        ),
    )()
    rope_cos = jnp.broadcast_to(cos_c[None, :, None, :], (b, s, 1, HEAD_DIM))
    rope_sin = jnp.broadcast_to(sin_c[None, :, None, :], (b, s, 1, HEAD_DIM))

    attention_mask = jnp.ones((b, s), dtype=jnp.float32)
    return embeddings, attention_mask, rope_cos, rope_sin
